# baseline (speedup 1.0000x reference)
import functools

import jax
import jax.numpy as jnp
from jax import lax
from jax.experimental import pallas as pl
from jax.experimental.pallas import tpu as pltpu
from jax.experimental.pallas import tpu_sc as plsc

N = 65536
NNZ = 4194304
D = 64
L = 16
NQ = D // L
NC = 2
NS = 16
CHUNK = 128
KB = 2
B = KB * CHUNK
NSETS = 8
GA = 6
EPT = NNZ // NS
NBATCH = EPT // B
RPT = N // NS
ZROWS = 256


def _sc_body(rows_hbm, cols_hbm, vals_hbm, tab_hbm, out_hbm,
             acc, rows_v, qidx_v, vals_v, gath, zbuf, zidx,
             sem_in, sem_g, sem_s):
    c = lax.axis_index("c")
    s = lax.axis_index("s")

    @pl.loop(0, ZROWS)
    def _(i):
        zbuf[i, :] = jnp.zeros((L,), jnp.float32)

    @pl.loop(0, CHUNK // L)
    def _(i):
        zidx[pl.ds(i * L, L)] = jnp.zeros((L,), jnp.int32)

    def in_descs(bi, t):
        base = s * (EPT // CHUNK) + t * KB
        return [
            pltpu.make_async_copy(rows_hbm.at[pl.ds(base, KB)], rows_v[bi],
                                  sem_in[bi]),
            pltpu.make_async_copy(cols_hbm.at[pl.ds(base, KB)], qidx_v[bi],
                                  sem_in[bi]),
            pltpu.make_async_copy(vals_hbm.at[pl.ds(base * CHUNK, B)],
                                  vals_v[bi], sem_in[bi]),
        ]

    def fire_in(bi, t):
        for d_ in in_descs(bi, t):
            d_.start()

    def drain_in(bi, t):
        for d_ in in_descs(bi, t):
            d_.wait()

    def qadd(bi, qoff):
        @plsc.parallel_loop(0, B // L, 1, unroll=8)
        def _(i):
            j = i // (CHUNK // L)
            k = (i % (CHUNK // L)) * L
            sl = pl.ds(k, L)
            qidx_v[bi][j, sl] = qidx_v[bi][j, sl] + qoff

    def g_descs(bi):
        return [
            pltpu.make_async_copy(tab_hbm.at[qidx_v[bi].at[j]],
                                  gath[bi].at[pl.ds(j * CHUNK, CHUNK)],
                                  sem_g[bi])
            for j in range(KB)
        ]

    def mult(bi):
        @plsc.parallel_loop(0, B // L, 1, unroll=2)
        def _(g):
            vv = vals_v[bi][pl.ds(g * L, L)]
            for t in range(L):
                e = g * L + t
                gath[bi][e, :] = gath[bi][e, :] * vv[t]

    def fire_s(bi):
        for j in range(KB):
            pltpu.async_copy(gath[bi].at[pl.ds(j * CHUNK, CHUNK)],
                             acc.at[rows_v[bi].at[j]], sem_s[bi], add=True)

    def drain_s(bi):
        for j in range(KB):
            pltpu.make_async_copy(gath[bi].at[pl.ds(j * CHUNK, CHUNK)],
                                  acc.at[rows_v[bi].at[j]], sem_s[bi]).wait()

    for qi in range(NQ // NC):
        q = c + NC * qi
        qoff = q * N

        for z in range(RPT // ZROWS):
            pltpu.sync_copy(zbuf, acc.at[pl.ds(s * RPT + z * ZROWS, ZROWS)])
        plsc.subcore_barrier()

        for bi in range(NSETS - 1):
            fire_in(bi, bi)
        for bi in range(GA):
            drain_in(bi, bi)
            qadd(bi, qoff)
            for d_ in g_descs(bi):
                d_.start()
        for j in range(KB):
            pltpu.async_copy(zbuf.at[pl.ds(0, CHUNK)], acc.at[zidx],
                             sem_s[NSETS - 1], add=True)

        @pl.loop(0, NBATCH, step=NSETS)
        def _(b):
            for r in range(NSETS):
                t = b + r
                ra, rb = (r + GA) % NSETS, (r + NSETS - 1) % NSETS
                for d_ in g_descs(r):
                    d_.wait()
                mult(r)
                fire_s(r)
                tg = jnp.minimum(t + GA, NBATCH - 1)
                drain_in(ra, tg)
                qadd(ra, qoff)
                for d_ in g_descs(ra):
                    d_.start()
                drain_s(rb)
                fire_in(rb, jnp.minimum(t + NSETS - 1, NBATCH - 1))

        for bi in range(GA):
            for d_ in g_descs(bi):
                d_.wait()
        for bi in range(GA, NSETS - 1):
            drain_in(bi, NBATCH - 1)
        drain_s(NSETS - 1)

        plsc.subcore_barrier()
        pltpu.sync_copy(acc.at[pl.ds(s * RPT, RPT)],
                        out_hbm.at[pl.ds(qoff + s * RPT, RPT)])
        plsc.subcore_barrier()


def kernel(adj_indices, adj_values, embeds):
    rows = adj_indices[0].astype(jnp.int32).reshape(NNZ // CHUNK, CHUNK)
    cols = adj_indices[1].astype(jnp.int32).reshape(NNZ // CHUNK, CHUNK)
    vals = adj_values.astype(jnp.float32)
    tab = embeds.astype(jnp.float32).reshape(N, NQ, L)
    tab = tab.transpose(1, 0, 2).reshape(NQ * N, L)

    spmm = pl.kernel(
        _sc_body,
        out_type=jax.ShapeDtypeStruct((NQ * N, L), jnp.float32),
        mesh=plsc.VectorSubcoreMesh(core_axis_name="c", subcore_axis_name="s"),
        scratch_types=[
            pltpu.VMEM_SHARED((N, L), jnp.float32),
            [pltpu.VMEM((KB, CHUNK), jnp.int32)] * NSETS,
            [pltpu.VMEM((KB, CHUNK), jnp.int32)] * NSETS,
            [pltpu.VMEM((B,), jnp.float32)] * NSETS,
            [pltpu.VMEM((B, L), jnp.float32)] * NSETS,
            pltpu.VMEM((ZROWS, L), jnp.float32),
            pltpu.VMEM((CHUNK,), jnp.int32),
            [pltpu.SemaphoreType.DMA] * NSETS,
            [pltpu.SemaphoreType.DMA] * NSETS,
            [pltpu.SemaphoreType.DMA] * NSETS,
        ],
        compiler_params=pltpu.CompilerParams(use_tc_tiling_on_sc=False),
    )
    out = spmm(rows, cols, vals, tab)
    return out.reshape(NQ, N, L).transpose(1, 0, 2).reshape(N, D)


# speedup vs baseline: 1.6008x; 1.6008x over previous
import jax
import jax.numpy as jnp
from jax import lax
from jax.experimental import pallas as pl
from jax.experimental.pallas import tpu as pltpu
from jax.experimental.pallas import tpu_sc as plsc

N = 65536
NNZ = 4194304
D = 64
L = 16
NQ = D // L
NC = 2
NS = 16
CHUNK = 128
KB = 2
B = KB * CHUNK
NSETS = 8
GA = 4
EPT = NNZ // NS
NBATCH = EPT // B
RPT = N // NS
ZROWS = 256


def _sc_body(rows_hbm, cols_hbm, vals_hbm, tab_hbm, out_hbm,
             acc, rows_v, qidx_v, vals_v, gath, zbuf, zidx,
             sem_in, sem_g, sem_s):
    c = lax.axis_index("c")
    s = lax.axis_index("s")

    @pl.loop(0, ZROWS)
    def _(i):
        zbuf[i, :] = jnp.zeros((L,), jnp.float32)

    @pl.loop(0, CHUNK // L)
    def _(i):
        zidx[pl.ds(i * L, L)] = jnp.zeros((L,), jnp.int32)

    def in_descs(bi, t):
        base = s * (EPT // CHUNK) + t * KB
        return [
            pltpu.make_async_copy(rows_hbm.at[pl.ds(base, KB)], rows_v[bi],
                                  sem_in[bi]),
            pltpu.make_async_copy(cols_hbm.at[pl.ds(base, KB)], qidx_v[bi],
                                  sem_in[bi]),
            pltpu.make_async_copy(vals_hbm.at[pl.ds(base * CHUNK, B)],
                                  vals_v[bi], sem_in[bi]),
        ]

    def fire_in(bi, t):
        for d_ in in_descs(bi, t):
            d_.start()

    def drain_in(bi, t):
        for d_ in in_descs(bi, t):
            d_.wait()

    def qadd(bi, qoff):
        @plsc.parallel_loop(0, B // L, 1, unroll=8)
        def _(i):
            j = i // (CHUNK // L)
            k = (i % (CHUNK // L)) * L
            sl = pl.ds(k, L)
            qidx_v[bi][j, sl] = qidx_v[bi][j, sl] + qoff

    def g_descs(bi):
        return [
            pltpu.make_async_copy(tab_hbm.at[qidx_v[bi].at[j]],
                                  gath[bi].at[pl.ds(j * CHUNK, CHUNK)],
                                  sem_g[bi])
            for j in range(KB)
        ]

    def mult(bi):
        @plsc.parallel_loop(0, B // L, 1, unroll=2)
        def _(g):
            vv = vals_v[bi][pl.ds(g * L, L)]
            for t in range(L):
                e = g * L + t
                gath[bi][e, :] = gath[bi][e, :] * vv[t]

    def fire_s(bi):
        for j in range(KB):
            pltpu.async_copy(gath[bi].at[pl.ds(j * CHUNK, CHUNK)],
                             acc.at[rows_v[bi].at[j]], sem_s[bi], add=True)

    def drain_s(bi):
        for j in range(KB):
            pltpu.make_async_copy(gath[bi].at[pl.ds(j * CHUNK, CHUNK)],
                                  acc.at[rows_v[bi].at[j]], sem_s[bi]).wait()

    for qi in range(NQ // NC):
        q = c + NC * qi
        qoff = q * N

        for z in range(RPT // ZROWS):
            pltpu.sync_copy(zbuf, acc.at[pl.ds(s * RPT + z * ZROWS, ZROWS)])
        plsc.subcore_barrier()

        for bi in range(NSETS - 1):
            fire_in(bi, bi)
        for bi in range(GA):
            drain_in(bi, bi)
            qadd(bi, qoff)
            for d_ in g_descs(bi):
                d_.start()
        for j in range(KB):
            pltpu.async_copy(zbuf.at[pl.ds(0, CHUNK)], acc.at[zidx],
                             sem_s[NSETS - 1], add=True)

        @pl.loop(0, NBATCH, step=NSETS)
        def _(b):
            for r in range(NSETS):
                t = b + r
                ra, rb = (r + GA) % NSETS, (r + NSETS - 1) % NSETS
                for d_ in g_descs(r):
                    d_.wait()
                mult(r)
                fire_s(r)
                tg = jnp.minimum(t + GA, NBATCH - 1)
                drain_in(ra, tg)
                qadd(ra, qoff)
                for d_ in g_descs(ra):
                    d_.start()
                drain_s(rb)
                fire_in(rb, jnp.minimum(t + NSETS - 1, NBATCH - 1))

        for bi in range(GA):
            for d_ in g_descs(bi):
                d_.wait()
        for bi in range(GA, NSETS - 1):
            drain_in(bi, NBATCH - 1)
        drain_s(NSETS - 1)

        plsc.subcore_barrier()
        pltpu.sync_copy(acc.at[pl.ds(s * RPT, RPT)],
                        out_hbm.at[pl.ds(qoff + s * RPT, RPT)])
        plsc.subcore_barrier()


def kernel(adj_indices, adj_values, embeds):
    rows = adj_indices[0].astype(jnp.int32).reshape(NNZ // CHUNK, CHUNK)
    cols = adj_indices[1].astype(jnp.int32).reshape(NNZ // CHUNK, CHUNK)
    vals = adj_values.astype(jnp.float32)
    tab = embeds.astype(jnp.float32).reshape(N, NQ, L)
    tab = tab.transpose(1, 0, 2).reshape(NQ * N, L)

    spmm = pl.kernel(
        _sc_body,
        out_type=jax.ShapeDtypeStruct((NQ * N, L), jnp.float32),
        mesh=plsc.VectorSubcoreMesh(core_axis_name="c", subcore_axis_name="s"),
        scratch_types=[
            pltpu.VMEM_SHARED((N, L), jnp.float32),
            [pltpu.VMEM((KB, CHUNK), jnp.int32)] * NSETS,
            [pltpu.VMEM((KB, CHUNK), jnp.int32)] * NSETS,
            [pltpu.VMEM((B,), jnp.float32)] * NSETS,
            [pltpu.VMEM((B, L), jnp.float32)] * NSETS,
            pltpu.VMEM((ZROWS, L), jnp.float32),
            pltpu.VMEM((CHUNK,), jnp.int32),
            [pltpu.SemaphoreType.DMA] * NSETS,
            [pltpu.SemaphoreType.DMA] * NSETS,
            [pltpu.SemaphoreType.DMA] * NSETS,
        ],
        compiler_params=pltpu.CompilerParams(use_tc_tiling_on_sc=False),
    )
    out = spmm(rows, cols, vals, tab)
    return out.reshape(NQ, N, L).transpose(1, 0, 2).reshape(N, D)
